# baseline (speedup 1.0000x reference)
import functools
import math

import jax
import jax.numpy as jnp
from jax.experimental import pallas as pl
from jax.experimental.pallas import tpu as pltpu


def _merge_ln_proj_kernel(x_ref, g_ref, b_ref, w_ref, o_ref, *, eps, cin):
    blk = x_ref[...]
    tq, _, wo, ch = blk.shape
    tokens = tq * wo
    top = blk[:, 0].reshape(tokens, ch)
    bot = blk[:, 1].reshape(tokens, ch)

    inv_cin = 1.0 / float(cin)
    mean = (jnp.sum(top, axis=-1, keepdims=True)
            + jnp.sum(bot, axis=-1, keepdims=True)) * inv_cin
    ct = top - mean
    cb = bot - mean
    var = (jnp.sum(ct * ct, axis=-1, keepdims=True)
           + jnp.sum(cb * cb, axis=-1, keepdims=True)) * inv_cin
    scale = jax.lax.rsqrt(var + eps)

    g = g_ref[...]
    b = b_ref[...]
    yt = (ct * scale * g[0:1, :] + b[0:1, :]).astype(jnp.bfloat16)
    yb = (cb * scale * g[1:2, :] + b[1:2, :]).astype(jnp.bfloat16)

    w = w_ref[...]
    acc = jnp.dot(yt, w[0], preferred_element_type=jnp.float32)
    acc = acc + jnp.dot(yb, w[1], preferred_element_type=jnp.float32)
    o_ref[...] = acc.astype(o_ref.dtype)


def kernel(x, gamma, beta, weight, *, eps=1e-5):
    B, L, C = x.shape
    H = W = math.isqrt(L)
    assert H * W == L and H % 2 == 0 and W % 2 == 0
    Ho, Wo = H // 2, W // 2
    Ch = 2 * C
    Cin = 4 * C
    Cout = weight.shape[0]
    Nq = B * Ho
    N = Nq * Wo
    out_dtype = x.dtype

    xr = x.reshape(Nq, 2, Wo, Ch)
    g2 = gamma.reshape(2, Ch)
    b2 = beta.reshape(2, Ch)
    w2 = weight.T.reshape(2, Ch, Cout).astype(jnp.bfloat16)

    tq = 32
    grid = (pl.cdiv(Nq, tq),)

    cost = pl.CostEstimate(
        flops=int(2 * N * Cin * Cout),
        transcendentals=int(N),
        bytes_accessed=int(N * Cin * x.dtype.itemsize
                           + N * Cout * jnp.dtype(out_dtype).itemsize
                           + Cin * Cout * 2),
    )

    out2d = pl.pallas_call(
        functools.partial(_merge_ln_proj_kernel, eps=eps, cin=Cin),
        out_shape=jax.ShapeDtypeStruct((N, Cout), out_dtype),
        grid=grid,
        in_specs=[
            pl.BlockSpec((tq, 2, Wo, Ch), lambda i: (i, 0, 0, 0)),
            pl.BlockSpec((2, Ch), lambda i: (0, 0)),
            pl.BlockSpec((2, Ch), lambda i: (0, 0)),
            pl.BlockSpec((2, Ch, Cout), lambda i: (0, 0, 0)),
        ],
        out_specs=pl.BlockSpec((tq * Wo, Cout), lambda i: (i, 0)),
        compiler_params=pltpu.CompilerParams(
            dimension_semantics=("parallel",),
            vmem_limit_bytes=64 * 2**20,
        ),
        cost_estimate=cost,
    )(xr, g2, b2, w2)

    return out2d.reshape(B, Ho * Wo, Cout)


# speedup vs baseline: 1.9087x; 1.9087x over previous
import functools
import math

import jax
import jax.numpy as jnp
from jax.experimental import pallas as pl
from jax.experimental.pallas import tpu as pltpu


def _merge_ln_proj_kernel(x_ref, g_ref, b_ref, w_ref, o_ref, *, eps, cin, c):
    blk = x_ref[...]
    tq, _, w_len, _ = blk.shape
    rows = tq * w_len

    p0 = blk[:, 0].reshape(rows // 2, 2 * c)
    p1 = blk[:, 1].reshape(rows // 2, 2 * c)
    e0, o0 = p0[:, :c], p0[:, c:]
    e1, o1 = p1[:, :c], p1[:, c:]

    inv_cin = 1.0 / float(cin)
    tot = (jnp.sum(e0, axis=-1, keepdims=True)
           + jnp.sum(o0, axis=-1, keepdims=True)
           + jnp.sum(e1, axis=-1, keepdims=True)
           + jnp.sum(o1, axis=-1, keepdims=True))
    mean = tot * inv_cin
    ce0, co0, ce1, co1 = e0 - mean, o0 - mean, e1 - mean, o1 - mean
    var = (jnp.sum(ce0 * ce0, axis=-1, keepdims=True)
           + jnp.sum(co0 * co0, axis=-1, keepdims=True)
           + jnp.sum(ce1 * ce1, axis=-1, keepdims=True)
           + jnp.sum(co1 * co1, axis=-1, keepdims=True)) * inv_cin
    scale = jax.lax.rsqrt(var + eps)

    g = g_ref[...]
    b = b_ref[...]
    w = w_ref[...].astype(jnp.bfloat16)

    dn = (((1,), (1,)), ((), ()))
    acc = None
    for k, cx in enumerate((ce0, co0, ce1, co1)):
        y = (cx * scale * g[:, k * c:(k + 1) * c]
             + b[:, k * c:(k + 1) * c]).astype(jnp.bfloat16)
        part = jax.lax.dot_general(y, w[:, k * c:(k + 1) * c], dn,
                                   preferred_element_type=jnp.float32)
        acc = part if acc is None else acc + part
    o_ref[...] = acc.astype(o_ref.dtype)


def kernel(x, gamma, beta, weight, *, eps=1e-5):
    B, L, C = x.shape
    H = W = math.isqrt(L)
    assert H * W == L and H % 2 == 0 and W % 2 == 0
    Ho, Wo = H // 2, W // 2
    Cin = 4 * C
    Cout = weight.shape[0]
    Nq = B * Ho
    N = Nq * Wo
    out_dtype = x.dtype

    xv = x.reshape(Nq, 2, W, C)
    g2 = gamma.reshape(1, Cin)
    b2 = beta.reshape(1, Cin)

    tq = 32
    grid = (pl.cdiv(Nq, tq),)

    cost = pl.CostEstimate(
        flops=int(2 * N * Cin * Cout),
        transcendentals=int(N),
        bytes_accessed=int(N * Cin * x.dtype.itemsize
                           + N * Cout * jnp.dtype(out_dtype).itemsize
                           + Cin * Cout * weight.dtype.itemsize),
    )

    out2d = pl.pallas_call(
        functools.partial(_merge_ln_proj_kernel, eps=eps, cin=Cin, c=C),
        out_shape=jax.ShapeDtypeStruct((N, Cout), out_dtype),
        grid=grid,
        in_specs=[
            pl.BlockSpec((tq, 2, W, C), lambda i: (i, 0, 0, 0)),
            pl.BlockSpec((1, Cin), lambda i: (0, 0)),
            pl.BlockSpec((1, Cin), lambda i: (0, 0)),
            pl.BlockSpec((Cout, Cin), lambda i: (0, 0)),
        ],
        out_specs=pl.BlockSpec((tq * Wo, Cout), lambda i: (i, 0)),
        compiler_params=pltpu.CompilerParams(
            dimension_semantics=("parallel",),
            vmem_limit_bytes=64 * 2**20,
        ),
        cost_estimate=cost,
    )(xv, g2, b2, weight)

    return out2d.reshape(B, Ho * Wo, Cout)


# speedup vs baseline: 2.1763x; 1.1402x over previous
import functools
import math

import jax
import jax.numpy as jnp
from jax.experimental import pallas as pl
from jax.experimental.pallas import tpu as pltpu

_NN = (((1,), (0,)), ((), ()))
_NT = (((1,), (1,)), ((), ()))
_F32 = jnp.float32
_BF16 = jnp.bfloat16


def _merge_ln_proj_kernel(x_ref, g_ref, b_ref, w_ref, o_ref, *, eps, cin, c):
    blk = x_ref[...]
    tq, _, w_len, _ = blk.shape
    rows = tq * w_len
    toks = rows // 2
    c2 = 2 * c

    w0 = blk[:, 0].astype(_BF16).reshape(toks, c2)
    w1 = blk[:, 1].astype(_BF16).reshape(toks, c2)

    ones = jnp.ones((c2, 1), _BF16)
    s = (jax.lax.dot_general(w0, ones, _NN, preferred_element_type=_F32)
         + jax.lax.dot_general(w1, ones, _NN, preferred_element_type=_F32))
    q = (jax.lax.dot_general(w0 * w0, ones, _NN, preferred_element_type=_F32)
         + jax.lax.dot_general(w1 * w1, ones, _NN, preferred_element_type=_F32))

    inv_cin = 1.0 / float(cin)
    mean = s * inv_cin
    var = q * inv_cin - mean * mean
    inv = jax.lax.rsqrt(var + eps)

    g = g_ref[...]
    b = b_ref[...]
    w = w_ref[...]
    wb = w.astype(_BF16)
    wp = (w * g).astype(_BF16)

    ones_row = jnp.ones((1, cin), _BF16)
    gw = jax.lax.dot_general(ones_row, wp, _NT, preferred_element_type=_F32)
    bw = jax.lax.dot_general(b.astype(_BF16), wb, _NT, preferred_element_type=_F32)

    u = None
    for k, src in enumerate((w0[:, :c], w0[:, c:], w1[:, :c], w1[:, c:])):
        part = jax.lax.dot_general(src, wp[:, k * c:(k + 1) * c], _NT,
                                   preferred_element_type=_F32)
        u = part if u is None else u + part

    o_ref[...] = ((u - mean * gw) * inv + bw).astype(o_ref.dtype)


def kernel(x, gamma, beta, weight, *, eps=1e-5):
    B, L, C = x.shape
    H = W = math.isqrt(L)
    assert H * W == L and H % 2 == 0 and W % 2 == 0
    Ho, Wo = H // 2, W // 2
    Cin = 4 * C
    Cout = weight.shape[0]
    Nq = B * Ho
    N = Nq * Wo
    out_dtype = x.dtype

    xv = x.reshape(Nq, 2, W, C)
    g2 = gamma.reshape(1, Cin)
    b2 = beta.reshape(1, Cin)

    tq = 32
    grid = (pl.cdiv(Nq, tq),)

    cost = pl.CostEstimate(
        flops=int(2 * N * Cin * Cout),
        transcendentals=int(N),
        bytes_accessed=int(N * Cin * x.dtype.itemsize
                           + N * Cout * jnp.dtype(out_dtype).itemsize
                           + Cin * Cout * weight.dtype.itemsize),
    )

    out2d = pl.pallas_call(
        functools.partial(_merge_ln_proj_kernel, eps=eps, cin=Cin, c=C),
        out_shape=jax.ShapeDtypeStruct((N, Cout), out_dtype),
        grid=grid,
        in_specs=[
            pl.BlockSpec((tq, 2, W, C), lambda i: (i, 0, 0, 0)),
            pl.BlockSpec((1, Cin), lambda i: (0, 0)),
            pl.BlockSpec((1, Cin), lambda i: (0, 0)),
            pl.BlockSpec((Cout, Cin), lambda i: (0, 0)),
        ],
        out_specs=pl.BlockSpec((tq * Wo, Cout), lambda i: (i, 0)),
        compiler_params=pltpu.CompilerParams(
            dimension_semantics=("parallel",),
            vmem_limit_bytes=64 * 2**20,
        ),
        cost_estimate=cost,
    )(xv, g2, b2, weight)

    return out2d.reshape(B, Ho * Wo, Cout)


# speedup vs baseline: 2.5374x; 1.1660x over previous
import functools
import math

import jax
import jax.numpy as jnp
from jax.experimental import pallas as pl
from jax.experimental.pallas import tpu as pltpu

_NN = (((1,), (0,)), ((), ()))
_NT = (((1,), (1,)), ((), ()))
_F32 = jnp.float32
_BF16 = jnp.bfloat16


def _merge_ln_proj_kernel(x_ref, g_ref, b_ref, w_ref, o_ref, *, eps, cin, c):
    blk = x_ref[...]
    tq, _, w_len, _ = blk.shape
    rows = tq * w_len
    toks = rows // 2
    c2 = 2 * c

    w0 = blk[:, 0].astype(_BF16).reshape(toks, c2)
    w1 = blk[:, 1].astype(_BF16).reshape(toks, c2)

    ones = jnp.ones((c2, 1), _BF16)
    s = (jax.lax.dot_general(w0, ones, _NN, preferred_element_type=_F32)
         + jax.lax.dot_general(w1, ones, _NN, preferred_element_type=_F32))
    q = (jax.lax.dot_general(w0 * w0, ones, _NN, preferred_element_type=_F32)
         + jax.lax.dot_general(w1 * w1, ones, _NN, preferred_element_type=_F32))

    inv_cin = 1.0 / float(cin)
    mean = s * inv_cin
    var = q * inv_cin - mean * mean
    inv = jax.lax.rsqrt(var + eps)

    g = g_ref[...]
    b = b_ref[...]
    w = w_ref[...]
    wb = w.astype(_BF16)
    wp = (w * g).astype(_BF16)

    ones_row = jnp.ones((1, cin), _BF16)
    gw = jax.lax.dot_general(ones_row, wp, _NT, preferred_element_type=_F32)
    bw = jax.lax.dot_general(b.astype(_BF16), wb, _NT, preferred_element_type=_F32)

    u = None
    for k, src in enumerate((w0[:, :c], w0[:, c:], w1[:, :c], w1[:, c:])):
        part = jax.lax.dot_general(src, wp[:, k * c:(k + 1) * c], _NT,
                                   preferred_element_type=_F32)
        u = part if u is None else u + part

    o_ref[...] = ((u - mean * gw) * inv + bw).astype(o_ref.dtype)


def kernel(x, gamma, beta, weight, *, eps=1e-5):
    B, L, C = x.shape
    H = W = math.isqrt(L)
    assert H * W == L and H % 2 == 0 and W % 2 == 0
    Ho, Wo = H // 2, W // 2
    Cin = 4 * C
    Cout = weight.shape[0]
    Nq = B * Ho
    N = Nq * Wo
    out_dtype = x.dtype

    xv = x.reshape(Nq, 2, W, C)
    g2 = gamma.reshape(1, Cin)
    b2 = beta.reshape(1, Cin)

    tq = 64
    grid = (pl.cdiv(Nq, tq),)

    cost = pl.CostEstimate(
        flops=int(2 * N * Cin * Cout),
        transcendentals=int(N),
        bytes_accessed=int(N * Cin * x.dtype.itemsize
                           + N * Cout * jnp.dtype(out_dtype).itemsize
                           + Cin * Cout * weight.dtype.itemsize),
    )

    out2d = pl.pallas_call(
        functools.partial(_merge_ln_proj_kernel, eps=eps, cin=Cin, c=C),
        out_shape=jax.ShapeDtypeStruct((N, Cout), out_dtype),
        grid=grid,
        in_specs=[
            pl.BlockSpec((tq, 2, W, C), lambda i: (i, 0, 0, 0)),
            pl.BlockSpec((1, Cin), lambda i: (0, 0)),
            pl.BlockSpec((1, Cin), lambda i: (0, 0)),
            pl.BlockSpec((Cout, Cin), lambda i: (0, 0)),
        ],
        out_specs=pl.BlockSpec((tq * Wo, Cout), lambda i: (i, 0)),
        compiler_params=pltpu.CompilerParams(
            dimension_semantics=("parallel",),
            vmem_limit_bytes=64 * 2**20,
        ),
        cost_estimate=cost,
    )(xv, g2, b2, weight)

    return out2d.reshape(B, Ho * Wo, Cout)


# speedup vs baseline: 2.5844x; 1.0185x over previous
import functools
import math

import jax
import jax.numpy as jnp
from jax.experimental import pallas as pl
from jax.experimental.pallas import tpu as pltpu

_NN = (((1,), (0,)), ((), ()))
_NT = (((1,), (1,)), ((), ()))
_F32 = jnp.float32
_BF16 = jnp.bfloat16


def _merge_ln_proj_kernel(x_ref, g_ref, b_ref, w_ref, o_ref, *, eps, cin, c):
    blk = x_ref[...]
    tq, _, w_len, _ = blk.shape
    rows = tq * w_len
    toks = rows // 2
    c2 = 2 * c

    w0 = blk[:, 0].astype(_BF16).reshape(toks, c2)
    w1 = blk[:, 1].astype(_BF16).reshape(toks, c2)

    ones = jnp.ones((c2, 1), _BF16)
    s = (jax.lax.dot_general(w0, ones, _NN, preferred_element_type=_F32)
         + jax.lax.dot_general(w1, ones, _NN, preferred_element_type=_F32))
    q = (jax.lax.dot_general(w0 * w0, ones, _NN, preferred_element_type=_F32)
         + jax.lax.dot_general(w1 * w1, ones, _NN, preferred_element_type=_F32))

    inv_cin = 1.0 / float(cin)
    mean = s * inv_cin
    var = q * inv_cin - mean * mean
    inv = jax.lax.rsqrt(var + eps)

    g = g_ref[...]
    b = b_ref[...]
    w = w_ref[...]
    wb = w.astype(_BF16)
    wp = (w * g).astype(_BF16)

    ones_row = jnp.ones((1, cin), _BF16)
    gw = jax.lax.dot_general(ones_row, wp, _NT, preferred_element_type=_F32)
    bw = jax.lax.dot_general(b.astype(_BF16), wb, _NT, preferred_element_type=_F32)

    u = None
    for k, src in enumerate((w0[:, :c], w0[:, c:], w1[:, :c], w1[:, c:])):
        part = jax.lax.dot_general(src, wp[:, k * c:(k + 1) * c], _NT,
                                   preferred_element_type=_F32)
        u = part if u is None else u + part

    o_ref[...] = ((u - mean * gw) * inv + bw).astype(o_ref.dtype)


def kernel(x, gamma, beta, weight, *, eps=1e-5):
    B, L, C = x.shape
    H = W = math.isqrt(L)
    assert H * W == L and H % 2 == 0 and W % 2 == 0
    Ho, Wo = H // 2, W // 2
    Cin = 4 * C
    Cout = weight.shape[0]
    Nq = B * Ho
    N = Nq * Wo
    out_dtype = x.dtype

    xv = x.reshape(Nq, 2, W, C)
    g2 = gamma.reshape(1, Cin)
    b2 = beta.reshape(1, Cin)

    tq = 128
    grid = (pl.cdiv(Nq, tq),)

    cost = pl.CostEstimate(
        flops=int(2 * N * Cin * Cout),
        transcendentals=int(N),
        bytes_accessed=int(N * Cin * x.dtype.itemsize
                           + N * Cout * jnp.dtype(out_dtype).itemsize
                           + Cin * Cout * weight.dtype.itemsize),
    )

    out2d = pl.pallas_call(
        functools.partial(_merge_ln_proj_kernel, eps=eps, cin=Cin, c=C),
        out_shape=jax.ShapeDtypeStruct((N, Cout), out_dtype),
        grid=grid,
        in_specs=[
            pl.BlockSpec((tq, 2, W, C), lambda i: (i, 0, 0, 0)),
            pl.BlockSpec((1, Cin), lambda i: (0, 0)),
            pl.BlockSpec((1, Cin), lambda i: (0, 0)),
            pl.BlockSpec((Cout, Cin), lambda i: (0, 0)),
        ],
        out_specs=pl.BlockSpec((tq * Wo, Cout), lambda i: (i, 0)),
        compiler_params=pltpu.CompilerParams(
            dimension_semantics=("parallel",),
            vmem_limit_bytes=64 * 2**20,
        ),
        cost_estimate=cost,
    )(xv, g2, b2, weight)

    return out2d.reshape(B, Ho * Wo, Cout)
